# baseline (speedup 1.0000x reference)
import functools

import jax
import jax.numpy as jnp
from jax.experimental import pallas as pl
from jax.experimental.pallas import tpu as pltpu


def _negd2_body(x_i_ref, x_j_ref, out_ref):
    xi = x_i_ref[...]
    xj = x_j_ref[...]
    g = jax.lax.dot_general(
        xi, xj, (((1,), (1,)), ((), ())), preferred_element_type=jnp.float32
    )
    sq_i = jnp.sum(xi * xi, axis=1)
    sq_j = jnp.sum(xj * xj, axis=1)
    out_ref[...] = 2.0 * g - sq_i[:, None] - sq_j[None, :]


@functools.partial(jax.jit, static_argnames=("bm", "bn"))
def _negd2(x, bm=512, bn=512):
    n, d = x.shape
    grid = (n // bm, n // bn)
    return pl.pallas_call(
        _negd2_body,
        grid=grid,
        in_specs=[
            pl.BlockSpec((bm, d), lambda i, j: (i, 0)),
            pl.BlockSpec((bn, d), lambda i, j: (j, 0)),
        ],
        out_specs=pl.BlockSpec((bm, bn), lambda i, j: (i, j)),
        out_shape=jax.ShapeDtypeStruct((n, n), jnp.float32),
    )(x, x)


def kernel(x, k):
    nd2 = _negd2(x)
    _, idx = jax.lax.top_k(nd2, 64)
    return idx.astype(jnp.int32) + (jnp.asarray(k, jnp.int32) - 64)


# speedup vs baseline: 9.2403x; 9.2403x over previous
import functools

import jax
import jax.numpy as jnp
from jax import lax
from jax.experimental import pallas as pl
from jax.experimental.pallas import tpu as pltpu
from jax.experimental.pallas import tpu_sc as plsc

BIG_F = 3.0e38


def _d2_body(x_i_ref, x_j_ref, d2_ref, g16_ref):
    xi = x_i_ref[...]
    xj = x_j_ref[...]
    g = lax.dot_general(
        xi, xj, (((1,), (1,)), ((), ())), preferred_element_type=jnp.float32
    )
    sq_i = jnp.sum(xi * xi, axis=1)
    sq_j = jnp.sum(xj * xj, axis=1)
    d2 = sq_i[:, None] + sq_j[None, :] - 2.0 * g
    d2_ref[...] = d2
    bm, bn = d2.shape
    g16_ref[...] = jnp.min(d2.reshape(bm, bn // 16, 16), axis=2)


@functools.partial(jax.jit, static_argnames=("bm", "bn"))
def _d2_and_g16(x, bm=256, bn=2048):
    n, d = x.shape
    grid = (n // bm, n // bn)
    return pl.pallas_call(
        _d2_body,
        grid=grid,
        in_specs=[
            pl.BlockSpec((bm, d), lambda i, j: (i, 0)),
            pl.BlockSpec((bn, d), lambda i, j: (j, 0)),
        ],
        out_specs=[
            pl.BlockSpec((bm, bn), lambda i, j: (i, j)),
            pl.BlockSpec((bm, bn // 16), lambda i, j: (i, j)),
        ],
        out_shape=[
            jax.ShapeDtypeStruct((n, n), jnp.float32),
            jax.ShapeDtypeStruct((n, n // 16), jnp.float32),
        ],
    )(x, x)


def _tree_min(v, iota):
    for s in (8, 4, 2, 1):
        v = jnp.minimum(v, jnp.take(v, (iota + s) % 16))
    return v


def _locate(v, key, iota):
    c = jnp.where(v == key, iota, 99)
    return _tree_min(c, iota)[0]


def _sc_topk_body(kk, d2_hbm, g16_hbm, out_hbm, row_v, g16_v, out_v, sem):
    del sem
    n = d2_hbm.shape[0]
    rows_per = n // 32
    wid = lax.axis_index("s") * 2 + lax.axis_index("c")
    row0 = wid * rows_per
    iota = lax.iota(jnp.int32, 16)

    def per_row(rr, _):
        row = row0 + rr
        pltpu.sync_copy(d2_hbm.at[row], row_v)
        pltpu.sync_copy(g16_hbm.at[row], g16_v)

        m2 = []
        for h in range(2):
            acc = jnp.full((16,), BIG_F, jnp.float32)
            for gg in range(16):
                mn = _tree_min(g16_v[pl.ds((h * 16 + gg) * 16, 16)], iota)[0]
                acc = jnp.where(iota == gg, mn, acc)
            m2.append(acc)
        m2a, m2b = m2

        def emit(tt, carry):
            m2a, m2b, acc = carry
            gkey = _tree_min(jnp.minimum(m2a, m2b), iota)[0]
            c1 = jnp.where(m2a == gkey, iota, 99)
            c2 = jnp.where(m2b == gkey, iota + 16, 99)
            sg = _tree_min(jnp.minimum(c1, c2), iota)[0]
            sv = g16_v[pl.ds(sg * 16, 16)]
            l1 = _locate(sv, gkey, iota)
            grp = sg * 16 + l1
            rv = row_v[pl.ds(grp * 16, 16)]
            l0 = _locate(rv, gkey, iota)
            acc = jnp.where(iota == tt, grp * 16 + l0, acc)
            rv2 = jnp.where(iota == l0, BIG_F, rv)
            row_v[pl.ds(grp * 16, 16)] = rv2
            nm1 = _tree_min(rv2, iota)[0]
            sv2 = jnp.where(iota == l1, nm1, sv)
            g16_v[pl.ds(sg * 16, 16)] = sv2
            nm2 = _tree_min(sv2, iota)[0]
            m2a = jnp.where(iota == sg, nm2, m2a)
            m2b = jnp.where(iota == sg - 16, nm2, m2b)
            return m2a, m2b, acc

        acc0 = jnp.zeros((16,), jnp.int32)
        carry = (m2a, m2b, acc0)
        for chunk in range(kk // 16):
            m2a, m2b, acc = lax.fori_loop(0, 16, emit, (carry[0], carry[1],
                                                        acc0))
            out_v[pl.ds(chunk * 16, 16)] = acc
            carry = (m2a, m2b, acc0)
        pltpu.sync_copy(out_v, out_hbm.at[row])
        return 0

    lax.fori_loop(0, rows_per, per_row, 0)


@functools.partial(jax.jit, static_argnames=("kk",))
def _sc_topk(d2, g16, kk=64):
    n = d2.shape[0]
    mesh = plsc.VectorSubcoreMesh(core_axis_name="c", subcore_axis_name="s",
                                  num_cores=2, num_subcores=16)
    return pl.kernel(
        functools.partial(_sc_topk_body, kk),
        out_type=jax.ShapeDtypeStruct((n, kk), jnp.int32),
        mesh=mesh,
        scratch_types=[
            pltpu.VMEM((n,), jnp.float32),
            pltpu.VMEM((n // 16,), jnp.float32),
            pltpu.VMEM((kk,), jnp.int32),
            pltpu.SemaphoreType.DMA,
        ],
    )(d2, g16)


def kernel(x, k):
    d2, g16 = _d2_and_g16(x)
    idx = _sc_topk(d2, g16)
    return idx + (jnp.asarray(k, jnp.int32) - 64)


# speedup vs baseline: 13.1657x; 1.4248x over previous
import functools

import jax
import jax.numpy as jnp
from jax import lax
from jax.experimental import pallas as pl
from jax.experimental.pallas import tpu as pltpu
from jax.experimental.pallas import tpu_sc as plsc

BIG_F = 3.0e38


def _d2_body(x_i_ref, x_j_ref, d2_ref, g16_ref):
    xi = x_i_ref[...]
    xj = x_j_ref[...]
    g = lax.dot_general(
        xi, xj, (((1,), (1,)), ((), ())), preferred_element_type=jnp.float32
    )
    sq_i = jnp.sum(xi * xi, axis=1)
    sq_j = jnp.sum(xj * xj, axis=1)
    d2 = sq_i[:, None] + sq_j[None, :] - 2.0 * g
    d2_ref[...] = d2
    bm, bn = d2.shape
    g16_ref[...] = jnp.min(d2.reshape(bm // 16, 16, bn), axis=1)


@functools.partial(jax.jit, static_argnames=("bm", "bn"))
def _d2_and_g16(x, bm=256, bn=2048):
    n, d = x.shape
    grid = (n // bm, n // bn)
    return pl.pallas_call(
        _d2_body,
        grid=grid,
        in_specs=[
            pl.BlockSpec((bm, d), lambda i, j: (i, 0)),
            pl.BlockSpec((bn, d), lambda i, j: (j, 0)),
        ],
        out_specs=[
            pl.BlockSpec((bm, bn), lambda i, j: (i, j)),
            pl.BlockSpec((bm // 16, bn), lambda i, j: (i, j)),
        ],
        out_shape=[
            jax.ShapeDtypeStruct((n, n), jnp.float32),
            jax.ShapeDtypeStruct((n // 16, n), jnp.float32),
        ],
    )(x, x)


def _tree_min(v, iota):
    for s in (8, 4, 2, 1):
        v = jnp.minimum(v, jnp.take(v, (iota + s) % 16))
    return v


def _locate(v, key, iota):
    c = jnp.where(v == key, iota, 99)
    return _tree_min(c, iota)[0]


def _sc_topk_body(kk, d2_hbm, g16_hbm, out_hbm, row_v, g16_v, out_v, sem):
    del sem
    n = d2_hbm.shape[0]
    rows_per = n // 32
    wid = lax.axis_index("s") * 2 + lax.axis_index("c")
    row0 = wid * rows_per
    iota = lax.iota(jnp.int32, 16)

    def per_row(rr, _):
        row = row0 + rr
        pltpu.sync_copy(d2_hbm.at[row], row_v)
        pltpu.sync_copy(g16_hbm.at[row], g16_v)

        m2 = []
        for h in range(2):
            acc = jnp.full((16,), BIG_F, jnp.float32)
            for gg in range(16):
                mn = _tree_min(g16_v[pl.ds((h * 16 + gg) * 16, 16)], iota)[0]
                acc = jnp.where(iota == gg, mn, acc)
            m2.append(acc)
        m2a, m2b = m2

        def emit(tt, carry):
            m2a, m2b, acc = carry
            gkey = _tree_min(jnp.minimum(m2a, m2b), iota)[0]
            c1 = jnp.where(m2a == gkey, iota, 99)
            c2 = jnp.where(m2b == gkey, iota + 16, 99)
            sg = _tree_min(jnp.minimum(c1, c2), iota)[0]
            sv = g16_v[pl.ds(sg * 16, 16)]
            l1 = _locate(sv, gkey, iota)
            grp = sg * 16 + l1
            rv = row_v[pl.ds(grp * 16, 16)]
            l0 = _locate(rv, gkey, iota)
            acc = jnp.where(iota == tt, grp * 16 + l0, acc)
            rv2 = jnp.where(iota == l0, BIG_F, rv)
            row_v[pl.ds(grp * 16, 16)] = rv2
            nm1 = _tree_min(rv2, iota)[0]
            sv2 = jnp.where(iota == l1, nm1, sv)
            g16_v[pl.ds(sg * 16, 16)] = sv2
            nm2 = _tree_min(sv2, iota)[0]
            m2a = jnp.where(iota == sg, nm2, m2a)
            m2b = jnp.where(iota == sg - 16, nm2, m2b)
            return m2a, m2b, acc

        acc0 = jnp.zeros((16,), jnp.int32)
        carry = (m2a, m2b, acc0)
        for chunk in range(kk // 16):
            m2a, m2b, acc = lax.fori_loop(0, 16, emit, (carry[0], carry[1],
                                                        acc0))
            out_v[pl.ds(chunk * 16, 16)] = acc
            carry = (m2a, m2b, acc0)
        pltpu.sync_copy(out_v, out_hbm.at[row])
        return 0

    lax.fori_loop(0, rows_per, per_row, 0)


@functools.partial(jax.jit, static_argnames=("kk",))
def _sc_topk(d2, g16, kk=64):
    n = d2.shape[0]
    mesh = plsc.VectorSubcoreMesh(core_axis_name="c", subcore_axis_name="s",
                                  num_cores=2, num_subcores=16)
    return pl.kernel(
        functools.partial(_sc_topk_body, kk),
        out_type=jax.ShapeDtypeStruct((n, kk), jnp.int32),
        mesh=mesh,
        scratch_types=[
            pltpu.VMEM((n,), jnp.float32),
            pltpu.VMEM((n // 16,), jnp.float32),
            pltpu.VMEM((kk,), jnp.int32),
            pltpu.SemaphoreType.DMA,
        ],
    )(d2, g16)


def kernel(x, k):
    d2, g16t = _d2_and_g16(x)
    idx = _sc_topk(d2, g16t.T)
    return idx + (jnp.asarray(k, jnp.int32) - 64)


# speedup vs baseline: 21.5152x; 1.6342x over previous
import functools

import jax
import jax.numpy as jnp
from jax import lax
from jax.experimental import pallas as pl
from jax.experimental.pallas import tpu as pltpu
from jax.experimental.pallas import tpu_sc as plsc

BIG_F = 3.0e38


def _d2_body(x_i_ref, x_j_ref, d2_ref, g16_ref):
    xi = x_i_ref[...]
    xj = x_j_ref[...]
    g = lax.dot_general(
        xi, xj, (((1,), (1,)), ((), ())), preferred_element_type=jnp.float32
    )
    sq_i = jnp.sum(xi * xi, axis=1)
    sq_j = jnp.sum(xj * xj, axis=1)
    d2 = sq_i[:, None] + sq_j[None, :] - 2.0 * g
    d2_ref[...] = d2
    bm, bn = d2.shape
    g16_ref[...] = jnp.min(d2.reshape(bm // 16, 16, bn), axis=1)


@functools.partial(jax.jit, static_argnames=("bm", "bn"))
def _d2_and_g16(x, bm=256, bn=2048):
    n, d = x.shape
    grid = (n // bm, n // bn)
    return pl.pallas_call(
        _d2_body,
        grid=grid,
        in_specs=[
            pl.BlockSpec((bm, d), lambda i, j: (i, 0)),
            pl.BlockSpec((bn, d), lambda i, j: (j, 0)),
        ],
        out_specs=[
            pl.BlockSpec((bm, bn), lambda i, j: (i, j)),
            pl.BlockSpec((bm // 16, bn), lambda i, j: (i, j)),
        ],
        out_shape=[
            jax.ShapeDtypeStruct((n, n), jnp.float32),
            jax.ShapeDtypeStruct((n // 16, n), jnp.float32),
        ],
    )(x, x)


def _tree_min(v, iota):
    for s in (8, 4, 2, 1):
        v = jnp.minimum(v, jnp.take(v, (iota + s) % 16))
    return v


def _locate(v, key, iota):
    c = jnp.where(v == key, iota, 99)
    return _tree_min(c, iota)[0]


def _build_m2(g16_ref, iota):
    m2 = []
    for h in range(2):
        acc = jnp.full((16,), BIG_F, jnp.float32)
        for gg in range(16):
            mn = _tree_min(g16_ref[pl.ds((h * 16 + gg) * 16, 16)], iota)[0]
            acc = jnp.where(iota == gg, mn, acc)
        m2.append(acc)
    return m2


def _emit_step(tt, m2a, m2b, acc, row_ref, g16_ref, iota):
    gkey = _tree_min(jnp.minimum(m2a, m2b), iota)[0]
    c1 = jnp.where(m2a == gkey, iota, 99)
    c2 = jnp.where(m2b == gkey, iota + 16, 99)
    sg = _tree_min(jnp.minimum(c1, c2), iota)[0]
    sv = g16_ref[pl.ds(sg * 16, 16)]
    l1 = _locate(sv, gkey, iota)
    grp = sg * 16 + l1
    rv = row_ref[pl.ds(grp * 16, 16)]
    l0 = _locate(rv, gkey, iota)
    acc = jnp.where(iota == tt, grp * 16 + l0, acc)
    rv2 = jnp.where(iota == l0, BIG_F, rv)
    row_ref[pl.ds(grp * 16, 16)] = rv2
    nm1 = _tree_min(rv2, iota)[0]
    sv2 = jnp.where(iota == l1, nm1, sv)
    g16_ref[pl.ds(sg * 16, 16)] = sv2
    nm2 = _tree_min(sv2, iota)[0]
    m2a = jnp.where(iota == sg, nm2, m2a)
    m2b = jnp.where(iota == sg - 16, nm2, m2b)
    return m2a, m2b, acc


def _sc_topk_body(kk, d2_hbm, g16_hbm, out_hbm,
                  row_a, row_b, g16_a, g16_b, out_a, out_b, sem_a, sem_b):
    n = d2_hbm.shape[0]
    rows_per = n // 32
    wid = lax.axis_index("s") * 2 + lax.axis_index("c")
    row0 = wid * rows_per
    iota = lax.iota(jnp.int32, 16)
    nlast = n - 1

    def copies(pair_base, rows, g16s, sem):
        ra = jnp.minimum(pair_base, nlast)
        rb = jnp.minimum(pair_base + 1, nlast)
        return (
            pltpu.make_async_copy(d2_hbm.at[ra], rows[0], sem),
            pltpu.make_async_copy(d2_hbm.at[rb], rows[1], sem),
            pltpu.make_async_copy(g16_hbm.at[ra], g16s[0], sem),
            pltpu.make_async_copy(g16_hbm.at[rb], g16s[1], sem),
        )

    def start(pair_base, rows, g16s, sem):
        for c in copies(pair_base, rows, g16s, sem):
            c.start()

    def wait(pair_base, rows, g16s, sem):
        for c in copies(pair_base, rows, g16s, sem):
            c.wait()

    def process_pair(pair_base, rows, g16s, outs):
        m2a0, m2b0 = _build_m2(g16s[0], iota)
        m2a1, m2b1 = _build_m2(g16s[1], iota)
        acc0 = jnp.zeros((16,), jnp.int32)

        def emit2(tt, carry):
            m2a0, m2b0, a0, m2a1, m2b1, a1 = carry
            m2a0, m2b0, a0 = _emit_step(tt, m2a0, m2b0, a0, rows[0], g16s[0],
                                        iota)
            m2a1, m2b1, a1 = _emit_step(tt, m2a1, m2b1, a1, rows[1], g16s[1],
                                        iota)
            return m2a0, m2b0, a0, m2a1, m2b1, a1

        carry = (m2a0, m2b0, acc0, m2a1, m2b1, acc0)
        for chunk in range(kk // 16):
            out = lax.fori_loop(0, 16, emit2,
                                (carry[0], carry[1], acc0, carry[3], carry[4],
                                 acc0))
            outs[0][pl.ds(chunk * 16, 16)] = out[2]
            outs[1][pl.ds(chunk * 16, 16)] = out[5]
            carry = out
        pltpu.sync_copy(outs[0], out_hbm.at[pair_base])
        pltpu.sync_copy(outs[1], out_hbm.at[pair_base + 1])

    set_a = ((row_a.at[0], row_a.at[1]), (g16_a.at[0], g16_a.at[1]),
             (out_a.at[0], out_a.at[1]), sem_a)
    set_b = ((row_b.at[0], row_b.at[1]), (g16_b.at[0], g16_b.at[1]),
             (out_b.at[0], out_b.at[1]), sem_b)

    start(row0, set_a[0], set_a[1], set_a[3])

    def quad(q, _):
        base_a = row0 + 4 * q
        base_b = base_a + 2
        start(base_b, set_b[0], set_b[1], set_b[3])
        wait(base_a, set_a[0], set_a[1], set_a[3])
        process_pair(base_a, set_a[0], set_a[1], set_a[2])
        start(base_a + 4, set_a[0], set_a[1], set_a[3])
        wait(base_b, set_b[0], set_b[1], set_b[3])
        process_pair(base_b, set_b[0], set_b[1], set_b[2])
        return 0

    lax.fori_loop(0, rows_per // 4, quad, 0)
    wait(row0 + rows_per, set_a[0], set_a[1], set_a[3])


@functools.partial(jax.jit, static_argnames=("kk",))
def _sc_topk(d2, g16, kk=64):
    n = d2.shape[0]
    mesh = plsc.VectorSubcoreMesh(core_axis_name="c", subcore_axis_name="s",
                                  num_cores=2, num_subcores=16)
    return pl.kernel(
        functools.partial(_sc_topk_body, kk),
        out_type=jax.ShapeDtypeStruct((n, kk), jnp.int32),
        mesh=mesh,
        scratch_types=[
            pltpu.VMEM((2, n), jnp.float32),
            pltpu.VMEM((2, n), jnp.float32),
            pltpu.VMEM((2, n // 16), jnp.float32),
            pltpu.VMEM((2, n // 16), jnp.float32),
            pltpu.VMEM((2, kk), jnp.int32),
            pltpu.VMEM((2, kk), jnp.int32),
            pltpu.SemaphoreType.DMA,
            pltpu.SemaphoreType.DMA,
        ],
    )(d2, g16)


def kernel(x, k):
    d2, g16t = _d2_and_g16(x)
    idx = _sc_topk(d2, g16t.T)
    return idx + (jnp.asarray(k, jnp.int32) - 64)
